# baseline (speedup 1.0000x reference)
import functools

import jax
import jax.numpy as jnp
from jax import lax
from jax.experimental import pallas as pl
from jax.experimental.pallas import tpu as pltpu
from jax.experimental.pallas import tpu_sc as plsc

_NUM_CLASSES = 100000
_EMB_DIM = 32
_BATCH = 16384

_NC = 2
_NS = 16
_LANES = 16
_NW = _NC * _NS
_ROWS_PER_W = _BATCH // _NW
_IDX_MINOR = 128
_IDX_CHUNKS = _ROWS_PER_W // _IDX_MINOR
_CHUNKS = _ROWS_PER_W // _LANES
_EPS = 1e-12


def _rsqrt_nr(x):
    i = lax.bitcast_convert_type(x, jnp.int32)
    i = jnp.int32(0x5F3759DF) - lax.shift_right_logical(i, 1)
    y = lax.bitcast_convert_type(i, jnp.float32)
    for _ in range(3):
        y = y * (jnp.float32(1.5) - jnp.float32(0.5) * x * y * y)
    return y


def _body(graph_hbm, labels_hbm, table_hbm, out_hbm, idx_v, g_v, rows_v, acc_v, sem):
    wid = lax.axis_index("s") * _NC + lax.axis_index("c")
    base = wid * _ROWS_PER_W

    g_cp = pltpu.make_async_copy(graph_hbm.at[pl.ds(base, _ROWS_PER_W)], g_v, sem)
    g_cp.start()
    pltpu.sync_copy(labels_hbm.at[wid], idx_v)
    gathers = []
    for j in range(_IDX_CHUNKS):
        cp = pltpu.make_async_copy(
            table_hbm.at[idx_v.at[j]],
            rows_v.at[pl.ds(j * _IDX_MINOR, _IDX_MINOR)],
            sem,
        )
        cp.start()
        gathers.append(cp)
    g_cp.wait()
    for cp in gathers:
        cp.wait()

    def chunk(c, acc):
        row_ids = c * _LANES + lax.iota(jnp.int32, _LANES)
        dot = jnp.zeros((_LANES,), jnp.float32)
        gg = jnp.zeros((_LANES,), jnp.float32)
        tt = jnp.zeros((_LANES,), jnp.float32)
        for d in range(_EMB_DIM):
            col = jnp.full((_LANES,), d, jnp.int32)
            t = plsc.load_gather(rows_v, [row_ids, col])
            g = plsc.load_gather(g_v, [row_ids, col])
            dot = dot + g * t
            gg = gg + g * g
            tt = tt + t * t
        norm_g = gg * _rsqrt_nr(gg)
        norm_t = tt * _rsqrt_nr(tt)
        inv = jnp.float32(1.0) / jnp.maximum(norm_g, jnp.float32(_EPS))
        inv = inv / jnp.maximum(norm_t, jnp.float32(_EPS))
        return acc + dot * inv

    acc = lax.fori_loop(0, _CHUNKS, chunk, jnp.zeros((_LANES,), jnp.float32))
    acc_v[...] = acc
    pltpu.sync_copy(acc_v, out_hbm.at[wid])


_sc_call = functools.partial(
    pl.kernel,
    out_type=jax.ShapeDtypeStruct((_NW, _LANES), jnp.float32),
    mesh=plsc.VectorSubcoreMesh(core_axis_name="c", subcore_axis_name="s"),
    compiler_params=pltpu.CompilerParams(
        needs_layout_passes=False, use_tc_tiling_on_sc=False
    ),
    scratch_types=[
        pltpu.VMEM((_IDX_CHUNKS, _IDX_MINOR), jnp.int32),
        pltpu.VMEM((_ROWS_PER_W, _EMB_DIM), jnp.float32),
        pltpu.VMEM((_ROWS_PER_W, _EMB_DIM), jnp.float32),
        pltpu.VMEM((_LANES,), jnp.float32),
        pltpu.SemaphoreType.DMA,
    ],
)(_body)


def kernel(graph_emb, labels, text_table):
    idx = labels.astype(jnp.int32).reshape(_NW, _IDX_CHUNKS, _IDX_MINOR)
    partials = _sc_call(graph_emb, idx, text_table)
    return (jnp.float32(1.0) - jnp.sum(partials) / jnp.float32(_BATCH)).astype(
        jnp.float32
    )


# speedup vs baseline: 1.0780x; 1.0780x over previous
import functools

import jax
import jax.numpy as jnp
from jax import lax
from jax.experimental import pallas as pl
from jax.experimental.pallas import tpu as pltpu
from jax.experimental.pallas import tpu_sc as plsc

_NUM_CLASSES = 100000
_EMB_DIM = 32
_BATCH = 16384

_NC = 2
_NS = 16
_LANES = 16
_NW = _NC * _NS
_ROWS_PER_W = _BATCH // _NW
_IDX_MINOR = 128
_IDX_CHUNKS = _ROWS_PER_W // _IDX_MINOR
_CHUNKS = _ROWS_PER_W // _LANES
_CHUNKS_PER_PHASE = _CHUNKS // _IDX_CHUNKS
_TROW = 128
_EPS = 1e-12


def _rsqrt_nr(x):
    i = lax.bitcast_convert_type(x, jnp.int32)
    i = jnp.int32(0x5F3759DF) - lax.shift_right_logical(i, 1)
    y = lax.bitcast_convert_type(i, jnp.float32)
    for _ in range(3):
        y = y * (jnp.float32(1.5) - jnp.float32(0.5) * x * y * y)
    return y


def _body(g_hbm, lab_hbm, tab_hbm, out_hbm,
          lab_v, sidx_v, g_v, rows_v, acc_v, sem_g, sem0, sem1, sem2, sem3):
    wid = lax.axis_index("s") * _NC + lax.axis_index("c")
    base = wid * _ROWS_PER_W

    g_cp = pltpu.make_async_copy(
        g_hbm.at[:, pl.ds(base, _ROWS_PER_W)], g_v, sem_g
    )
    g_cp.start()
    pltpu.sync_copy(lab_hbm.at[pl.ds(base, _ROWS_PER_W)], lab_v)
    sems = [sem0, sem1, sem2, sem3]
    gathers = []
    for j in range(_IDX_CHUNKS):
        for c in range(_IDX_MINOR // _LANES):
            o = j * _IDX_MINOR + c * _LANES
            sidx_v[pl.ds(o, _LANES)] = lax.shift_right_logical(
                lab_v[pl.ds(o, _LANES)], 2
            )
        cp = pltpu.make_async_copy(
            tab_hbm.at[sidx_v.at[pl.ds(j * _IDX_MINOR, _IDX_MINOR)]],
            rows_v.at[pl.ds(j * _IDX_MINOR, _IDX_MINOR)],
            sems[j],
        )
        cp.start()
        gathers.append(cp)
    g_cp.wait()

    iota = lax.iota(jnp.int32, _LANES)

    def chunk(c, acc):
        r_vec = c * _LANES + iota
        lm_vec = (lab_v[pl.ds(c * _LANES, _LANES)] & 3) << 5
        d_vec = iota
        dot = jnp.zeros((_LANES,), jnp.float32)
        gg = jnp.zeros((_LANES,), jnp.float32)
        tt = jnp.zeros((_LANES,), jnp.float32)
        for s in range(_EMB_DIM):
            t = plsc.load_gather(rows_v, [r_vec, lm_vec + d_vec])
            g = plsc.load_gather(g_v, [d_vec, r_vec])
            dot = dot + g * t
            gg = gg + g * g
            tt = tt + t * t
            if s < _EMB_DIM - 1:
                d_vec = (d_vec + 1) & (_EMB_DIM - 1)
        norm_g = gg * _rsqrt_nr(gg)
        norm_t = tt * _rsqrt_nr(tt)
        inv = jnp.float32(1.0) / jnp.maximum(norm_g, jnp.float32(_EPS))
        inv = inv / jnp.maximum(norm_t, jnp.float32(_EPS))
        return acc + dot * inv

    acc = jnp.zeros((_LANES,), jnp.float32)
    for j in range(_IDX_CHUNKS):
        gathers[j].wait()
        acc = lax.fori_loop(
            j * _CHUNKS_PER_PHASE, (j + 1) * _CHUNKS_PER_PHASE, chunk, acc
        )
    acc_v[...] = acc
    pltpu.sync_copy(acc_v, out_hbm.at[wid])


_sc_call = functools.partial(
    pl.kernel,
    out_type=jax.ShapeDtypeStruct((_NW, _LANES), jnp.float32),
    mesh=plsc.VectorSubcoreMesh(core_axis_name="c", subcore_axis_name="s"),
    compiler_params=pltpu.CompilerParams(
        needs_layout_passes=False, use_tc_tiling_on_sc=True
    ),
    scratch_types=[
        pltpu.VMEM((_ROWS_PER_W,), jnp.int32),
        pltpu.VMEM((_ROWS_PER_W,), jnp.int32),
        pltpu.VMEM((_EMB_DIM, _ROWS_PER_W), jnp.float32),
        pltpu.VMEM((_ROWS_PER_W, _TROW), jnp.float32),
        pltpu.VMEM((_LANES,), jnp.float32),
        pltpu.SemaphoreType.DMA,
        pltpu.SemaphoreType.DMA,
        pltpu.SemaphoreType.DMA,
        pltpu.SemaphoreType.DMA,
        pltpu.SemaphoreType.DMA,
    ],
)(_body)


def kernel(graph_emb, labels, text_table):
    g_t = graph_emb.T
    table2 = text_table.reshape(_NUM_CLASSES // 4, _TROW)
    lab = labels.astype(jnp.int32)
    partials = _sc_call(g_t, lab, table2)
    return (jnp.float32(1.0) - jnp.sum(partials) / jnp.float32(_BATCH)).astype(
        jnp.float32
    )


# speedup vs baseline: 1.1122x; 1.0317x over previous
import functools

import jax
import jax.numpy as jnp
from jax import lax
from jax.experimental import pallas as pl
from jax.experimental.pallas import tpu as pltpu
from jax.experimental.pallas import tpu_sc as plsc

_NUM_CLASSES = 100000
_EMB_DIM = 32
_BATCH = 16384

_NC = 2
_NS = 16
_LANES = 16
_NW = _NC * _NS
_ROWS_PER_W = _BATCH // _NW
_IDX_MINOR = 128
_IDX_CHUNKS = _ROWS_PER_W // _IDX_MINOR
_CHUNKS = _ROWS_PER_W // _LANES
_CHUNKS_PER_PHASE = _CHUNKS // _IDX_CHUNKS
_EPS = 1e-12

def _rsqrt_nr(x):
    i = lax.bitcast_convert_type(x, jnp.int32)
    i = jnp.int32(0x5F3759DF) - lax.shift_right_logical(i, 1)
    y = lax.bitcast_convert_type(i, jnp.float32)
    for _ in range(3):
        y = y * (jnp.float32(1.5) - jnp.float32(0.5) * x * y * y)
    return y


def _body(g_hbm, lab_hbm, tab_hbm, out_hbm,
          lab_v, g_v, rows_v, acc_v, sem_g, sem0, sem1, sem2, sem3):
    wid = lax.axis_index("s") * _NC + lax.axis_index("c")
    base = wid * _ROWS_PER_W

    g_cp = pltpu.make_async_copy(
        g_hbm.at[:, pl.ds(base, _ROWS_PER_W)], g_v, sem_g
    )
    g_cp.start()
    pltpu.sync_copy(lab_hbm.at[pl.ds(base, _ROWS_PER_W)], lab_v)
    sems = [sem0, sem1, sem2, sem3]
    gathers = []
    for j in range(_IDX_CHUNKS):
        cp = pltpu.make_async_copy(
            tab_hbm.at[lab_v.at[pl.ds(j * _IDX_MINOR, _IDX_MINOR)]],
            rows_v.at[pl.ds(j * _IDX_MINOR, _IDX_MINOR), :],
            sems[j],
        )
        cp.start()
        gathers.append(cp)
    g_cp.wait()

    iota = lax.iota(jnp.int32, _LANES)

    def chunk(c, acc):
        r_vec = c * _LANES + iota
        d_vec = iota
        dot = jnp.zeros((_LANES,), jnp.float32)
        gg = jnp.zeros((_LANES,), jnp.float32)
        tt = jnp.zeros((_LANES,), jnp.float32)
        for s in range(_EMB_DIM):
            t = plsc.load_gather(rows_v, [r_vec, d_vec])
            g = plsc.load_gather(g_v, [d_vec, r_vec])
            dot = dot + g * t
            gg = gg + g * g
            tt = tt + t * t
            if s < _EMB_DIM - 1:
                d_vec = (d_vec + 1) & (_EMB_DIM - 1)
        norm_g = gg * _rsqrt_nr(gg)
        norm_t = tt * _rsqrt_nr(tt)
        inv = jnp.float32(1.0) / jnp.maximum(norm_g, jnp.float32(_EPS))
        inv = inv / jnp.maximum(norm_t, jnp.float32(_EPS))
        return acc + dot * inv

    acc = jnp.zeros((_LANES,), jnp.float32)
    for j in range(_IDX_CHUNKS):
        gathers[j].wait()
        acc = lax.fori_loop(
            j * _CHUNKS_PER_PHASE, (j + 1) * _CHUNKS_PER_PHASE, chunk, acc
        )
    acc_v[...] = acc
    pltpu.sync_copy(acc_v, out_hbm.at[wid])


_sc_call = functools.partial(
    pl.kernel,
    out_type=jax.ShapeDtypeStruct((_NW, _LANES), jnp.float32),
    mesh=plsc.VectorSubcoreMesh(core_axis_name="c", subcore_axis_name="s"),
    compiler_params=pltpu.CompilerParams(
        needs_layout_passes=False, use_tc_tiling_on_sc=False
    ),
    scratch_types=[
        pltpu.VMEM((_ROWS_PER_W,), jnp.int32),
        pltpu.VMEM((_EMB_DIM, _ROWS_PER_W), jnp.float32),
        pltpu.VMEM((_ROWS_PER_W, _EMB_DIM), jnp.float32),
        pltpu.VMEM((_LANES,), jnp.float32),
        pltpu.SemaphoreType.DMA,
        pltpu.SemaphoreType.DMA,
        pltpu.SemaphoreType.DMA,
        pltpu.SemaphoreType.DMA,
        pltpu.SemaphoreType.DMA,
    ],
)(_body)


def kernel(graph_emb, labels, text_table):
    lab = labels.astype(jnp.int32)
    partials = _sc_call(graph_emb.T, lab, text_table)
    return (jnp.float32(1.0) - jnp.sum(partials) / jnp.float32(_BATCH)).astype(
        jnp.float32
    )
